# Initial kernel scaffold; baseline (speedup 1.0000x reference)
import jax
import jax.numpy as jnp
from jax.experimental import pallas as pl


def kernel(center_words, target_words, all_vocabs, emb_v, emb_u):
    raise NotImplementedError("write your pallas kernel here")


# speedup vs baseline: 50.0666x; 50.0666x over previous
import functools

import jax
import jax.numpy as jnp
from jax import lax
from jax.experimental import pallas as pl
from jax.experimental.pallas import tpu as pltpu
from jax.experimental.pallas import tpu_sc as plsc

B = 1024
V = 1000
VP = 1024
E = 64
NC = 2
NS = 16
L = 16
NW = NC * NS
ROWS = B // NW


def _tc_scores_body(center_ref, target_ref, emb_v_ref, emb_u_t_ref,
                    e_out_ref, scores_ref):
    col = lax.broadcasted_iota(jnp.int32, (B, VP), 1)
    oh_c = (center_ref[...] == col).astype(jnp.float32)
    c = jnp.dot(oh_c, emb_v_ref[...], preferred_element_type=jnp.float32)
    m = jnp.dot(c, emb_u_t_ref[...], preferred_element_type=jnp.float32)
    e_out_ref[...] = jnp.where(col < V, jnp.exp(m), 0.0)
    scores_ref[...] = jnp.sum(jnp.where(target_ref[...] == col, m, 0.0),
                              axis=1, keepdims=True)


_tc_scores = pl.pallas_call(
    _tc_scores_body,
    out_shape=(
        jax.ShapeDtypeStruct((B, VP), jnp.float32),
        jax.ShapeDtypeStruct((B, 1), jnp.float32),
    ),
)


_sc_mesh = plsc.VectorSubcoreMesh(core_axis_name="c", subcore_axis_name="s")


@functools.partial(
    pl.kernel,
    out_type=jax.ShapeDtypeStruct((B,), jnp.float32),
    mesh=_sc_mesh,
    compiler_params=pltpu.CompilerParams(
        use_tc_tiling_on_sc=False, needs_layout_passes=False),
    scratch_types=[
        pltpu.VMEM((ROWS, VP), jnp.float32),
        pltpu.VMEM((ROWS, VP), jnp.int32),
        pltpu.VMEM((ROWS,), jnp.float32),
    ],
)
def _sc_gather_sum(e_hbm, idx_hbm, out_hbm, e_v, idx_v, sum_v):
    wid = lax.axis_index("s") * NC + lax.axis_index("c")
    base = wid * ROWS
    pltpu.sync_copy(e_hbm.at[pl.ds(base, ROWS)], e_v)
    pltpu.sync_copy(idx_hbm.at[pl.ds(base, ROWS)], idx_v)
    for g in range(ROWS // L):
        rows = lax.iota(jnp.int32, L) + (g * L)

        def body(j, acc, _rows=rows):
            jv = jnp.zeros((L,), jnp.int32) + j
            cols = plsc.load_gather(idx_v, [_rows, jv])
            return acc + plsc.load_gather(e_v, [_rows, cols])

        acc = lax.fori_loop(0, VP, body, jnp.zeros((L,), jnp.float32))
        sum_v[pl.ds(g * L, L)] = acc
    pltpu.sync_copy(sum_v, out_hbm.at[pl.ds(base, ROWS)])


def _tc_nll_body(scores_ref, ns_ref, out_ref):
    nll = jnp.mean(jnp.log(ns_ref[...])) - jnp.mean(scores_ref[...])
    out_ref[...] = jnp.broadcast_to(nll, (1, 1))


_tc_nll = pl.pallas_call(
    _tc_nll_body,
    out_shape=jax.ShapeDtypeStruct((1, 1), jnp.float32),
)


def kernel(center_words, target_words, all_vocabs, emb_v, emb_u):
    emb_v_p = jnp.pad(emb_v, ((0, VP - V), (0, 0)))
    emb_u_t = jnp.pad(emb_u, ((0, VP - V), (0, 0))).T
    idx_p = jnp.pad(all_vocabs, ((0, 0), (0, VP - V)), constant_values=V)
    e, scores = _tc_scores(center_words, target_words, emb_v_p, emb_u_t)
    norm_sum = _sc_gather_sum(e, idx_p)
    nll = _tc_nll(scores, norm_sum.reshape(B, 1))
    return nll[0, 0]


# speedup vs baseline: 66.4972x; 1.3282x over previous
import functools

import jax
import jax.numpy as jnp
from jax import lax
from jax.experimental import pallas as pl
from jax.experimental.pallas import tpu as pltpu
from jax.experimental.pallas import tpu_sc as plsc

B = 1024
V = 1000
E = 64
NC = 2
NS = 16
L = 16
NW = NC * NS
ROWS = B // NW


def _tc_scores_body(center_ref, target_ref, emb_v_ref, emb_u_ref,
                    e_out_ref, scores_ref):
    col = lax.broadcasted_iota(jnp.int32, (B, V), 1)
    oh_c = (center_ref[...] == col).astype(jnp.float32)
    c = jnp.dot(oh_c, emb_v_ref[...], preferred_element_type=jnp.float32)
    m = lax.dot_general(c, emb_u_ref[...], (((1,), (1,)), ((), ())),
                        preferred_element_type=jnp.float32)
    e_out_ref[...] = jnp.exp(m)
    scores_ref[...] = jnp.sum(jnp.where(target_ref[...] == col, m, 0.0),
                              axis=1, keepdims=True)


_tc_scores = pl.pallas_call(
    _tc_scores_body,
    out_shape=(
        jax.ShapeDtypeStruct((B, V), jnp.float32),
        jax.ShapeDtypeStruct((B, 1), jnp.float32),
    ),
)


_sc_mesh = plsc.VectorSubcoreMesh(core_axis_name="c", subcore_axis_name="s")


@functools.partial(
    pl.kernel,
    out_type=jax.ShapeDtypeStruct((B,), jnp.float32),
    mesh=_sc_mesh,
    compiler_params=pltpu.CompilerParams(
        use_tc_tiling_on_sc=False, needs_layout_passes=False),
    scratch_types=[
        pltpu.VMEM((ROWS, V), jnp.float32),
        pltpu.VMEM((ROWS, V), jnp.int32),
        pltpu.VMEM((ROWS,), jnp.float32),
        pltpu.SemaphoreType.DMA,
        pltpu.SemaphoreType.DMA,
    ],
)
def _sc_gather_sum(e_hbm, idx_hbm, out_hbm, e_v, idx_v, sum_v, sem_e, sem_i):
    wid = lax.axis_index("s") * NC + lax.axis_index("c")
    base = wid * ROWS
    cp_e = pltpu.async_copy(e_hbm.at[pl.ds(base, ROWS)], e_v, sem_e)
    cp_i = pltpu.async_copy(idx_hbm.at[pl.ds(base, ROWS)], idx_v, sem_i)
    cp_e.wait()
    cp_i.wait()
    for g in range(ROWS // L):
        rows = lax.iota(jnp.int32, L) + (g * L)

        def body(j, acc, _rows=rows):
            jv = jnp.zeros((L,), jnp.int32) + j
            cols = plsc.load_gather(idx_v, [_rows, jv])
            return acc + plsc.load_gather(e_v, [_rows, cols])

        acc = lax.fori_loop(0, V, body, jnp.zeros((L,), jnp.float32),
                            unroll=8)
        sum_v[pl.ds(g * L, L)] = acc
    pltpu.sync_copy(sum_v, out_hbm.at[pl.ds(base, ROWS)])


def _tc_nll_body(scores_ref, ns_ref, out_ref):
    nll = jnp.mean(jnp.log(ns_ref[...])) - jnp.mean(scores_ref[...])
    out_ref[...] = jnp.broadcast_to(nll, (1, 1))


_tc_nll = pl.pallas_call(
    _tc_nll_body,
    out_shape=jax.ShapeDtypeStruct((1, 1), jnp.float32),
)


def kernel(center_words, target_words, all_vocabs, emb_v, emb_u):
    e, scores = _tc_scores(center_words, target_words, emb_v, emb_u)
    norm_sum = _sc_gather_sum(e, all_vocabs)
    return _tc_nll(scores, norm_sum)[0, 0]
